# Initial kernel scaffold; baseline (speedup 1.0000x reference)
import jax
import jax.numpy as jnp
from jax.experimental import pallas as pl


def kernel(x, Ws, bs, Wr, br, Wg, bg, gate_bias):
    raise NotImplementedError("write your pallas kernel here")


# speedup vs baseline: 1.9480x; 1.9480x over previous
import functools

import jax
import jax.numpy as jnp
from jax.experimental import pallas as pl

D = 1024
E = 8
TOP_K = 2
TN = 256


def _moe_block_kernel(x_ref, WsT_ref, bs_ref, WrT_ref, br_ref, WgT_ref,
                      bgt_ref, out_ref):
    x = x_ref[...]

    scores = jnp.dot(x, WgT_ref[...],
                     preferred_element_type=jnp.float32) + bgt_ref[...]
    neg_inf = jnp.float32(-jnp.inf)
    v1 = jnp.max(scores, axis=-1, keepdims=True)
    eidx = jax.lax.broadcasted_iota(jnp.int32, scores.shape, 1)
    a1 = jnp.min(jnp.where(scores == v1, eidx, E), axis=-1, keepdims=True)
    h1 = eidx == a1
    scores2 = jnp.where(h1, neg_inf, scores)
    v2 = jnp.max(scores2, axis=-1, keepdims=True)
    a2 = jnp.min(jnp.where(scores2 == v2, eidx, E), axis=-1, keepdims=True)
    h2 = eidx == a2
    w1 = jax.nn.sigmoid(v1 - v2)
    w2 = 1.0 - w1
    gates = h1 * w1 + h2 * w2

    acc = jnp.dot(x, WsT_ref[...],
                  preferred_element_type=jnp.float32) + bs_ref[...] + x

    for e in range(E):
        ye = jnp.dot(x, WrT_ref[e],
                     preferred_element_type=jnp.float32) + br_ref[e]
        acc = acc + gates[:, e:e + 1] * ye

    out_ref[...] = jnp.maximum(acc, 0.0)


@jax.jit
def kernel(x, Ws, bs, Wr, br, Wg, bg, gate_bias):
    N = x.shape[0]
    WsT = Ws.T
    WrT = jnp.swapaxes(Wr, 1, 2)
    WgT = Wg.T
    bs2 = bs.reshape(1, D)
    br2 = br.reshape(E, 1, D)
    bgt = (bg + gate_bias).reshape(1, E)

    grid = (N // TN,)
    out = pl.pallas_call(
        _moe_block_kernel,
        grid=grid,
        in_specs=[
            pl.BlockSpec((TN, D), lambda i: (i, 0)),
            pl.BlockSpec((D, D), lambda i: (0, 0)),
            pl.BlockSpec((1, D), lambda i: (0, 0)),
            pl.BlockSpec((E, D, D), lambda i: (0, 0, 0)),
            pl.BlockSpec((E, 1, D), lambda i: (0, 0, 0)),
            pl.BlockSpec((D, E), lambda i: (0, 0)),
            pl.BlockSpec((1, E), lambda i: (0, 0)),
        ],
        out_specs=pl.BlockSpec((TN, D), lambda i: (i, 0)),
        out_shape=jax.ShapeDtypeStruct((N, D), jnp.float32),
    )(x, WsT, bs2, WrT, br2, WgT, bgt)
    return out


# speedup vs baseline: 2.1842x; 1.1213x over previous
import functools

import jax
import jax.numpy as jnp
from jax.experimental import pallas as pl

D = 1024
E = 8
TOP_K = 2
TN = 256


def _moe_block_kernel(x_ref, WsT_ref, bs_ref, WrT_ref, br_ref, WgT_ref,
                      bgt_ref, out_ref):
    x = x_ref[...]
    xb = x.astype(jnp.bfloat16)

    scores = jnp.dot(x, WgT_ref[...],
                     preferred_element_type=jnp.float32) + bgt_ref[...]
    neg_inf = jnp.float32(-jnp.inf)
    v1 = jnp.max(scores, axis=-1, keepdims=True)
    eidx = jax.lax.broadcasted_iota(jnp.int32, scores.shape, 1)
    a1 = jnp.min(jnp.where(scores == v1, eidx, E), axis=-1, keepdims=True)
    h1 = eidx == a1
    scores2 = jnp.where(h1, neg_inf, scores)
    v2 = jnp.max(scores2, axis=-1, keepdims=True)
    a2 = jnp.min(jnp.where(scores2 == v2, eidx, E), axis=-1, keepdims=True)
    h2 = eidx == a2
    w1 = jax.nn.sigmoid(v1 - v2)
    w2 = 1.0 - w1
    gates = h1 * w1 + h2 * w2

    acc = jnp.dot(xb, WsT_ref[...],
                  preferred_element_type=jnp.float32) + bs_ref[...] + x

    for e in range(E):
        ye = jnp.dot(xb, WrT_ref[e],
                     preferred_element_type=jnp.float32) + br_ref[e]
        acc = acc + gates[:, e:e + 1] * ye

    out_ref[...] = jnp.maximum(acc, 0.0)


@jax.jit
def kernel(x, Ws, bs, Wr, br, Wg, bg, gate_bias):
    N = x.shape[0]
    WsT = Ws.T.astype(jnp.bfloat16)
    WrT = jnp.swapaxes(Wr, 1, 2).astype(jnp.bfloat16)
    WgT = Wg.T
    bs2 = bs.reshape(1, D)
    br2 = br.reshape(E, 1, D)
    bgt = (bg + gate_bias).reshape(1, E)

    grid = (N // TN,)
    out = pl.pallas_call(
        _moe_block_kernel,
        grid=grid,
        in_specs=[
            pl.BlockSpec((TN, D), lambda i: (i, 0)),
            pl.BlockSpec((D, D), lambda i: (0, 0)),
            pl.BlockSpec((1, D), lambda i: (0, 0)),
            pl.BlockSpec((E, D, D), lambda i: (0, 0, 0)),
            pl.BlockSpec((E, 1, D), lambda i: (0, 0, 0)),
            pl.BlockSpec((D, E), lambda i: (0, 0)),
            pl.BlockSpec((1, E), lambda i: (0, 0)),
        ],
        out_specs=pl.BlockSpec((TN, D), lambda i: (i, 0)),
        out_shape=jax.ShapeDtypeStruct((N, D), jnp.float32),
    )(x, WsT, bs2, WrT, br2, WgT, bgt)
    return out


# speedup vs baseline: 2.7598x; 1.2635x over previous
import jax
import jax.numpy as jnp
from jax import lax
from jax.experimental import pallas as pl

D = 1024
E = 8
TOP_K = 2
TN = 256

_DN_T = (((1,), (1,)), ((), ()))


def _moe_block_kernel(x_ref, Ws_ref, bs_ref, Wr_ref, br_ref, Wg_ref,
                      bg_ref, gb_ref, out_ref):
    x = x_ref[...]

    scores = lax.dot_general(
        x, Wg_ref[...], _DN_T,
        preferred_element_type=jnp.float32) + bg_ref[...] + gb_ref[...]
    neg_inf = jnp.float32(-jnp.inf)
    v1 = jnp.max(scores, axis=-1, keepdims=True)
    eidx = lax.broadcasted_iota(jnp.int32, scores.shape, 1)
    a1 = jnp.min(jnp.where(scores == v1, eidx, E), axis=-1, keepdims=True)
    h1 = eidx == a1
    scores2 = jnp.where(h1, neg_inf, scores)
    v2 = jnp.max(scores2, axis=-1, keepdims=True)
    a2 = jnp.min(jnp.where(scores2 == v2, eidx, E), axis=-1, keepdims=True)
    h2 = eidx == a2
    w1 = jax.nn.sigmoid(v1 - v2)
    w2 = 1.0 - w1
    gates = h1 * w1 + h2 * w2

    acc = lax.dot_general(x, Ws_ref[...], _DN_T,
                          preferred_element_type=jnp.float32) + bs_ref[...] + x

    for e in range(E):
        ye = lax.dot_general(x, Wr_ref[e], _DN_T,
                             preferred_element_type=jnp.float32) + br_ref[e]
        acc = acc + gates[:, e:e + 1] * ye

    out_ref[...] = jnp.maximum(acc, 0.0)


@jax.jit
def kernel(x, Ws, bs, Wr, br, Wg, bg, gate_bias):
    N = x.shape[0]
    bs2 = bs.reshape(1, D)
    br2 = br.reshape(E, 1, D)
    bg2 = bg.reshape(1, E)
    gb2 = gate_bias.reshape(1, E)

    grid = (N // TN,)
    out = pl.pallas_call(
        _moe_block_kernel,
        grid=grid,
        in_specs=[
            pl.BlockSpec((TN, D), lambda i: (i, 0)),
            pl.BlockSpec((D, D), lambda i: (0, 0)),
            pl.BlockSpec((1, D), lambda i: (0, 0)),
            pl.BlockSpec((E, D, D), lambda i: (0, 0, 0)),
            pl.BlockSpec((E, 1, D), lambda i: (0, 0, 0)),
            pl.BlockSpec((E, D), lambda i: (0, 0)),
            pl.BlockSpec((1, E), lambda i: (0, 0)),
            pl.BlockSpec((1, E), lambda i: (0, 0)),
        ],
        out_specs=pl.BlockSpec((TN, D), lambda i: (i, 0)),
        out_shape=jax.ShapeDtypeStruct((N, D), jnp.float32),
    )(x, Ws, bs2, Wr, br2, Wg, bg2, gb2)
    return out
